# baseline (speedup 1.0000x reference)
import functools

import jax
import jax.numpy as jnp
from jax import lax
from jax.experimental import pallas as pl
from jax.experimental.pallas import tpu as pltpu
from jax.experimental.pallas import tpu_sc as plsc

B = 16384
D = 64
LANES = 16
NUM_CORES = 2
NUM_SUBCORES = 16
NW = NUM_CORES * NUM_SUBCORES
BPW = B // NW
IDX_CHUNK = 128
NCHUNK = BPW // IDX_CHUNK


def _sc_partials(z, idx3, off3, centers2):
    mesh = plsc.VectorSubcoreMesh(core_axis_name="c", subcore_axis_name="s")

    @functools.partial(
        pl.kernel,
        out_type=jax.ShapeDtypeStruct((NW, LANES), jnp.float32),
        mesh=mesh,
        scratch_types=[
            pltpu.VMEM((NCHUNK, IDX_CHUNK), jnp.int32),
            pltpu.VMEM((BPW,), jnp.int32),
            pltpu.VMEM((BPW, 2 * D), jnp.float32),
            pltpu.VMEM((BPW // 2, 2 * D), jnp.float32),
            pltpu.VMEM((LANES,), jnp.float32),
            pltpu.SemaphoreType.DMA,
        ],
    )
    def k(z_hbm, idx_hbm, off_hbm, centers_hbm, out_hbm,
          idx_v, off_v, c_v, z_v, acc_v, sem):
        wid = lax.axis_index("s") * NUM_CORES + lax.axis_index("c")

        pltpu.sync_copy(idx_hbm.at[wid], idx_v)
        gathers = [
            pltpu.async_copy(
                centers_hbm.at[idx_v.at[j]],
                c_v.at[pl.ds(j * IDX_CHUNK, IDX_CHUNK)],
                sem,
            )
            for j in range(NCHUNK)
        ]
        pltpu.sync_copy(off_hbm.at[wid], off_v)
        pltpu.sync_copy(z_hbm.at[wid], z_v)
        for g in gathers:
            g.wait()

        def body(g, accs):
            offv = off_v[pl.ds(g * LANES, LANES)]
            accs = list(accs)
            for r in range(LANES):
                i = g * LANES + r
                zrow = g * (LANES // 2) + r // 2
                zcol = (r % 2) * D
                off = offv[r]
                for j in range(D // LANES):
                    dz = (z_v[zrow, pl.ds(zcol + j * LANES, LANES)]
                          - c_v[i, pl.ds(off + j * LANES, LANES)])
                    accs[j] = accs[j] + dz * dz
            return tuple(accs)

        zero = jnp.zeros((LANES,), jnp.float32)
        accs = lax.fori_loop(0, BPW // LANES, body, (zero,) * (D // LANES))
        acc_v[...] = accs[0] + accs[1] + accs[2] + accs[3]
        pltpu.sync_copy(acc_v, out_hbm.at[wid])

    return k(z, idx3, off3, centers2)


def _reduce_partials(partials):
    def body(p_ref, o_ref):
        o_ref[0, 0] = jnp.sum(p_ref[...]) * (1.0 / B)

    out = pl.pallas_call(
        body,
        out_shape=jax.ShapeDtypeStruct((1, 1), jnp.float32),
        out_specs=pl.BlockSpec(memory_space=pltpu.SMEM),
    )(partials)
    return out[0, 0]


def kernel(z, labels, centers):
    labels = labels.astype(jnp.int32)
    idx3 = (labels >> 1).reshape(NW, NCHUNK, IDX_CHUNK)
    off3 = ((labels & 1) * D).reshape(NW, BPW)
    centers2 = centers.reshape(centers.shape[0] // 2, 2 * D)
    z2 = z.reshape(NW, BPW // 2, 2 * D)
    partials = _sc_partials(z2, idx3, off3, centers2)
    return _reduce_partials(partials)


# speedup vs baseline: 2.5446x; 2.5446x over previous
import functools

import jax
import jax.numpy as jnp
from jax import lax
from jax.experimental import pallas as pl
from jax.experimental.pallas import tpu as pltpu
from jax.experimental.pallas import tpu_sc as plsc

B = 16384
D = 64
LANES = 16
SUBL = 8
NUM_CORES = 2
NUM_SUBCORES = 16
NW = NUM_CORES * NUM_SUBCORES
BPW = B // NW
NGRP = BPW // LANES


def _sc_partials(z4, blk2, sub2, centers3):
    mesh = plsc.VectorSubcoreMesh(core_axis_name="c", subcore_axis_name="s")

    @functools.partial(
        pl.kernel,
        out_type=jax.ShapeDtypeStruct((NW, LANES), jnp.float32),
        mesh=mesh,
        scratch_types=[
            pltpu.VMEM((BPW,), jnp.int32),
            pltpu.VMEM((BPW,), jnp.int32),
            pltpu.VMEM((BPW // 2, 2 * D), jnp.float32),
            pltpu.VMEM((BPW // SUBL, SUBL, D), jnp.float32),
            pltpu.VMEM((LANES,), jnp.float32),
            pltpu.SemaphoreType.DMA,
        ],
    )
    def k(z_hbm, blk_hbm, sub_hbm, centers_hbm, out_hbm,
          blk_v, sub_v, c_v, z_v, acc_v, sem):
        wid = lax.axis_index("s") * NUM_CORES + lax.axis_index("c")

        pltpu.sync_copy(blk_hbm.at[wid], blk_v)
        pltpu.sync_copy(sub_hbm.at[wid], sub_v)

        def issue(g, carry):
            blkv = blk_v[pl.ds(g * LANES, LANES)]
            subv = sub_v[pl.ds(g * LANES, LANES)]
            for r in range(LANES):
                pltpu.async_copy(
                    centers_hbm.at[blkv[r], subv[r]],
                    c_v.at[g * (LANES // 2) + r // 2, pl.ds((r % 2) * D, D)],
                    sem,
                )
            return carry

        lax.fori_loop(0, NGRP, issue, 0)
        pltpu.sync_copy(z_hbm.at[wid], z_v)

        def body(g, accs):
            accs = list(accs)
            for r in range(LANES):
                pltpu.make_async_copy(
                    centers_hbm.at[0, 0],
                    c_v.at[g * (LANES // 2) + r // 2, pl.ds((r % 2) * D, D)],
                    sem).wait()
            for r in range(LANES):
                crow = g * (LANES // 2) + r // 2
                ccol = (r % 2) * D
                zblk = g * (LANES // SUBL) + r // SUBL
                zsub = r % SUBL
                for j in range(D // LANES):
                    dz = (z_v[zblk, zsub, pl.ds(j * LANES, LANES)]
                          - c_v[crow, pl.ds(ccol + j * LANES, LANES)])
                    accs[j] = accs[j] + dz * dz
            return tuple(accs)

        zero = jnp.zeros((LANES,), jnp.float32)
        accs = lax.fori_loop(0, NGRP, body, (zero,) * (D // LANES))
        acc_v[...] = accs[0] + accs[1] + accs[2] + accs[3]
        pltpu.sync_copy(acc_v, out_hbm.at[wid])

    return k(z4, blk2, sub2, centers3)


def _reduce_partials(partials):
    def body(p_ref, o_ref):
        o_ref[0, 0] = jnp.sum(p_ref[...]) * (1.0 / B)

    out = pl.pallas_call(
        body,
        out_shape=jax.ShapeDtypeStruct((1, 1), jnp.float32),
        out_specs=pl.BlockSpec(memory_space=pltpu.SMEM),
    )(partials)
    return out[0, 0]


def kernel(z, labels, centers):
    labels = labels.astype(jnp.int32)
    blk2 = (labels >> 3).reshape(NW, BPW)
    sub2 = (labels & 7).reshape(NW, BPW)
    centers3 = centers.reshape(centers.shape[0] // SUBL, SUBL, D)
    z4 = z.reshape(NW, BPW // SUBL, SUBL, D)
    partials = _sc_partials(z4, blk2, sub2, centers3)
    return _reduce_partials(partials)
